# baseline (speedup 1.0000x reference)
import functools
import math

import jax
import jax.numpy as jnp
from jax import lax
from jax.experimental import pallas as pl
from jax.experimental.pallas import tpu as pltpu
from jax.experimental.pallas import tpu_sc as plsc

_NC = 2
_NS = 16
_NW = _NC * _NS

_DIM = 64
_SCALE = 8.0

_STREAM_ROWS = 128
_STREAMS_PER_CHUNK = 5
_CHUNK = _STREAM_ROWS * _STREAMS_PER_CHUNK


def _make_gather(n_total: int):
    assert n_total % (_NW * _CHUNK) == 0
    per_w = n_total // _NW
    n_chunks = per_w // _CHUNK

    mesh = plsc.VectorSubcoreMesh(
        core_axis_name="c", subcore_axis_name="s",
        num_cores=_NC, num_subcores=_NS,
    )

    @functools.partial(
        pl.kernel,
        out_type=jax.ShapeDtypeStruct((n_total, _DIM), jnp.float32),
        mesh=mesh,
        scratch_types=[
            pltpu.VMEM((per_w,), jnp.int32),
            pltpu.VMEM((_CHUNK, _DIM), jnp.float32),
            pltpu.VMEM((_CHUNK, _DIM), jnp.float32),
            pltpu.SemaphoreType.DMA,
            pltpu.SemaphoreType.DMA,
            pltpu.SemaphoreType.DMA,
            pltpu.SemaphoreType.DMA,
        ],
        compiler_params=pltpu.CompilerParams(use_tc_tiling_on_sc=False),
    )
    def emb_kernel(table_hbm, idx_hbm, out_hbm,
                   idx_v, rows0, rows1, g0, g1, s0, s1):
        wid = lax.axis_index("s") * _NC + lax.axis_index("c")
        base = wid * per_w
        rows = (rows0, rows1)
        gsem = (g0, g1)
        ssem = (s0, s1)

        pltpu.sync_copy(idx_hbm.at[pl.ds(base, per_w)], idx_v)

        def fire_gathers(t):
            buf = rows[t % 2]
            sem = gsem[t % 2]
            handles = []
            for j in range(_STREAMS_PER_CHUNK):
                s = t * _STREAMS_PER_CHUNK + j
                handles.append(pltpu.async_copy(
                    table_hbm.at[idx_v.at[pl.ds(s * _STREAM_ROWS,
                                                _STREAM_ROWS)]],
                    buf.at[pl.ds(j * _STREAM_ROWS, _STREAM_ROWS)],
                    sem,
                ))
            return handles

        def fire_store(t):
            buf = rows[t % 2]
            return pltpu.async_copy(
                buf, out_hbm.at[pl.ds(base + t * _CHUNK, _CHUNK)],
                ssem[t % 2],
            )

        pending_g = fire_gathers(0)
        pending_s = [None, None]
        for t in range(n_chunks):
            for h in pending_g:
                h.wait()
            if t + 1 < n_chunks:
                prev = pending_s[(t + 1) % 2]
                if prev is not None:
                    prev.wait()
                    pending_s[(t + 1) % 2] = None
                pending_g = fire_gathers(t + 1)
            pending_s[t % 2] = fire_store(t)
        for h in pending_s:
            if h is not None:
                h.wait()

    return emb_kernel


_PLANES_PER_BLOCK = 8


def _scale_reshape_block(g_ref, out_ref):
    seq = out_ref.shape[1]
    for p in range(_PLANES_PER_BLOCK):
        out_ref[p] = g_ref[pl.ds(p * seq, seq), :] * _SCALE


def _make_fixup(b: int, seq: int):
    n_blocks = b // _PLANES_PER_BLOCK
    rows_per_block = _PLANES_PER_BLOCK * seq
    return pl.pallas_call(
        _scale_reshape_block,
        grid=(n_blocks,),
        in_specs=[pl.BlockSpec((rows_per_block, _DIM),
                               lambda i: (i, 0))],
        out_specs=pl.BlockSpec((_PLANES_PER_BLOCK, seq, _DIM),
                               lambda i: (i, 0, 0)),
        out_shape=jax.ShapeDtypeStruct((b, seq, _DIM), jnp.float32),
    )


def kernel(input_vec, table):
    b, seq = input_vec.shape
    n_total = b * seq
    idx = input_vec.reshape(n_total).astype(jnp.int32)
    flat = _make_gather(n_total)(table, idx)
    return _make_fixup(b, seq)(flat)
